# Initial kernel scaffold; baseline (speedup 1.0000x reference)
import jax
import jax.numpy as jnp
from jax.experimental import pallas as pl


def kernel(x, wih0, whh0, b0, wih1, whh1, b1, wfc, bfc):
    raise NotImplementedError("write your pallas kernel here")


# speedup vs baseline: 2.4632x; 2.4632x over previous
import functools

import jax
import jax.numpy as jnp
from jax import lax
from jax.experimental import pallas as pl
from jax.experimental.pallas import tpu as pltpu

I_SIZE = 3
H = 64
OUTPAD = 128


def _gate_cols(w, layer):
    i, f, g, o = jnp.split(w, 4, axis=1)
    z = jnp.zeros_like(i)
    pairs = ((i, z), (f, z), (o, z), (g, z)) if layer == 0 else \
            ((z, i), (z, f), (z, o), (z, g))
    return jnp.concatenate([blk for pair in pairs for blk in pair], axis=1)


def _lstm_kernel(x_ref, wx_ref, wh_ref, bb_ref, wfc_ref, bfc_ref, out_ref,
                 pre, hseq, *, T, TB):
    pre[...] = jnp.dot(x_ref[...], wx_ref[...],
                       preferred_element_type=jnp.float32)

    wh = wh_ref[...]
    lane = lax.broadcasted_iota(jnp.int32, (TB, 2 * H), 1)
    l0_mask = lane < H

    def _advance(gsum, c):
        ifo = jax.nn.sigmoid(gsum[:, :6 * H])
        gg = jnp.tanh(gsum[:, 6 * H:])
        c = ifo[:, 2 * H:4 * H] * c + ifo[:, :2 * H] * gg
        h = ifo[:, 4 * H:6 * H] * jnp.tanh(c)
        return h, c

    h, c = _advance(pre[pl.ds(0, TB), :], jnp.zeros((TB, 2 * H), jnp.float32))
    h = jnp.where(l0_mask, h, 0.0)
    c = jnp.where(l0_mask, c, 0.0)

    def _step(s, carry):
        h, c = carry
        gsum = jnp.dot(h, wh, preferred_element_type=jnp.float32) \
            + pre[pl.ds(s * TB, TB), :]
        h, c = _advance(gsum, c)
        hseq[pl.ds((s - 1) * TB, TB), :] = h
        return h, c

    h, c = lax.fori_loop(1, T, _step, (h, c), unroll=2)

    gsum = jnp.dot(h, wh, preferred_element_type=jnp.float32) + bb_ref[...]
    h, _ = _advance(gsum, c)
    hseq[pl.ds((T - 1) * TB, TB), :] = h

    y = jnp.dot(hseq[...], wfc_ref[...],
                preferred_element_type=jnp.float32) + bfc_ref[...]
    out_ref[...] = y[:, :I_SIZE].astype(out_ref.dtype)


@jax.jit
def _forward(x, wih0, whh0, b0, wih1, whh1, b1, wfc, bfc):
    B, T, I = x.shape
    TB = 256
    if B < TB:
        TB = max(8, -(-B // 8) * 8)
    Bpad = -(-B // TB) * TB
    nb = Bpad // TB

    ones = jnp.ones((B, T, 1), x.dtype)
    xa = jnp.concatenate([x, ones], axis=2)
    xa = jnp.pad(xa, ((0, Bpad - B), (0, 0), (0, 0)))
    xa = xa.reshape(nb, TB, T, I + 1).transpose(0, 2, 1, 3)
    xa = xa.reshape(nb, T * TB, I + 1)

    bb = _gate_cols(b0, 0) + _gate_cols(b1, 1)
    wx = jnp.concatenate([_gate_cols(wih0, 0), bb], axis=0)
    wh = jnp.concatenate(
        [_gate_cols(whh0, 0) + _gate_cols(wih1, 1),
         _gate_cols(whh1, 1)], axis=0)
    wfcp = jnp.zeros((2 * H, OUTPAD), jnp.float32).at[H:, :I].set(wfc)
    bfcp = jnp.zeros((1, OUTPAD), jnp.float32).at[:, :I].set(bfc)

    out = pl.pallas_call(
        functools.partial(_lstm_kernel, T=T, TB=TB),
        out_shape=jax.ShapeDtypeStruct((nb, T * TB, I), x.dtype),
        grid=(nb,),
        in_specs=[
            pl.BlockSpec((None, T * TB, I + 1), lambda i: (i, 0, 0)),
            pl.BlockSpec((I + 1, 8 * H), lambda i: (0, 0)),
            pl.BlockSpec((2 * H, 8 * H), lambda i: (0, 0)),
            pl.BlockSpec((1, 8 * H), lambda i: (0, 0)),
            pl.BlockSpec((2 * H, OUTPAD), lambda i: (0, 0)),
            pl.BlockSpec((1, OUTPAD), lambda i: (0, 0)),
        ],
        out_specs=pl.BlockSpec((None, T * TB, I), lambda i: (i, 0, 0)),
        scratch_shapes=[
            pltpu.VMEM((T * TB, 8 * H), jnp.float32),
            pltpu.VMEM((T * TB, 2 * H), jnp.float32),
        ],
        compiler_params=pltpu.CompilerParams(
            dimension_semantics=("parallel",),
            vmem_limit_bytes=60 * 1024 * 1024,
        ),
    )(xa, wx, wh, bb, wfcp, bfcp)

    out = out.reshape(nb, T, TB, I).transpose(0, 2, 1, 3)
    return out.reshape(Bpad, T, I)[:B]


def kernel(x, wih0, whh0, b0, wih1, whh1, b1, wfc, bfc):
    return _forward(x, wih0, whh0, b0, wih1, whh1, b1, wfc, bfc)
